# Initial kernel scaffold; baseline (speedup 1.0000x reference)
import jax
import jax.numpy as jnp
from jax.experimental import pallas as pl


def kernel(data, true_branch_nodes, false_branch_nodes, start_index, exit_index, steps, embed, Wx, Wh, b, branch_W, branch_b, out_W, out_b):
    raise NotImplementedError("write your pallas kernel here")


# speedup vs baseline: 21.5374x; 21.5374x over previous
import functools

import jax
import jax.numpy as jnp
from jax.experimental import pallas as pl
from jax.experimental.pallas import tpu as pltpu

B, N, L = 8, 128, 4
VOCAB, OUT_VOCAB, H = 1000, 1000, 128
VOCAB_PAD = 1024
OUT_PAD = 1024
MAX_STEPS = 6
F32 = jnp.float32


def _dot(a, b):
    return jax.lax.dot(a, b, preferred_element_type=F32)


def _fwd_kernel(exit_ref, steps_ref, data_ref, tidx_ref, fidx_ref, embed_ref,
                wx_ref, wh_ref, b_ref, bw_ref, bb_ref, outw_ref, outb_ref,
                out_ref):
    pid = pl.program_id(0)
    exit_i = exit_ref[pid]
    num_steps = steps_ref[pid]

    wh = wh_ref[...]
    bias = b_ref[...]

    col_iota = jax.lax.broadcasted_iota(jnp.int32, (N, VOCAB_PAD), 1)
    xw = []
    for l in range(L):
        toks = data_ref[0, l, :]
        oh = (toks[:, None] == col_iota).astype(F32)
        emb_l = _dot(oh, embed_ref[...])
        xw.append(_dot(emb_l, wx_ref[...]) + bias)

    row_iota = jax.lax.broadcasted_iota(jnp.int32, (N, N), 0)
    mt = (tidx_ref[0] == row_iota).astype(F32)
    mf = (fidx_ref[0] == row_iota).astype(F32)

    node_iota = jax.lax.broadcasted_iota(jnp.int32, (N, 1), 0)
    exit_mask = node_iota == exit_i
    ones = jnp.ones((N, N), F32)

    c = jnp.zeros((N, H), F32)
    h = jnp.zeros((N, H), F32)
    ip = (node_iota == 0).astype(F32)

    for s in range(MAX_STEPS):
        cc, hh = c, h
        for l in range(L):
            z = xw[l] + _dot(hh, wh)
            i_g = jax.nn.sigmoid(z[:, :H])
            f_g = jax.nn.sigmoid(z[:, H:2 * H])
            g_g = jnp.tanh(z[:, 2 * H:3 * H])
            o_g = jax.nn.sigmoid(z[:, 3 * H:])
            cc = f_g * cc + i_g * g_g
            hh = o_g * jnp.tanh(cc)
        ce = jnp.where(exit_mask, c, cc)
        he = jnp.where(exit_mask, h, hh)
        hcat = jnp.concatenate([ce, he], axis=1)
        bl = _dot(hcat, bw_ref[...]) + bb_ref[...]
        p_true = jax.nn.sigmoid(bl[:, 0:1] - bl[:, 1:2])
        wt = p_true * ip
        wf = (1.0 - p_true) * ip
        a = jnp.concatenate([hcat, ones], axis=1)
        r = _dot(mt, wt * a) + _dot(mf, wf * a)
        ip_new = r[:, 2 * H:2 * H + 1]
        denom = ip_new + 1e-7
        keep = jnp.int32(s) < num_steps
        c = jnp.where(keep, r[:, :H] / denom, c)
        h = jnp.where(keep, r[:, H:2 * H] / denom, h)
        ip = jnp.where(keep, ip_new, ip)

    e_row = (jax.lax.broadcasted_iota(jnp.int32, (1, N), 1) == exit_i
             ).astype(F32)
    fin = jnp.concatenate([_dot(e_row, c), _dot(e_row, h)], axis=1)
    out_ref[0] = _dot(fin, outw_ref[...]) + outb_ref[...]


@functools.partial(jax.jit, static_argnames=())
def _forward_impl(data_t, tb, fb, exit_index, steps, embed_p, Wx, Wh, b2,
                  bW, bb2, outW_p, outb_p):
    grid_spec = pltpu.PrefetchScalarGridSpec(
        num_scalar_prefetch=2,
        grid=(B,),
        in_specs=[
            pl.BlockSpec((1, L, N), lambda i, *_: (i, 0, 0)),
            pl.BlockSpec((1, 1, N), lambda i, *_: (i, 0, 0)),
            pl.BlockSpec((1, 1, N), lambda i, *_: (i, 0, 0)),
            pl.BlockSpec((VOCAB_PAD, H), lambda i, *_: (0, 0)),
            pl.BlockSpec((H, 4 * H), lambda i, *_: (0, 0)),
            pl.BlockSpec((H, 4 * H), lambda i, *_: (0, 0)),
            pl.BlockSpec((1, 4 * H), lambda i, *_: (0, 0)),
            pl.BlockSpec((2 * H, 2), lambda i, *_: (0, 0)),
            pl.BlockSpec((1, 2), lambda i, *_: (0, 0)),
            pl.BlockSpec((2 * H, OUT_PAD), lambda i, *_: (0, 0)),
            pl.BlockSpec((1, OUT_PAD), lambda i, *_: (0, 0)),
        ],
        out_specs=pl.BlockSpec((1, 1, OUT_PAD), lambda i, *_: (i, 0, 0)),
    )
    out = pl.pallas_call(
        _fwd_kernel,
        grid_spec=grid_spec,
        out_shape=jax.ShapeDtypeStruct((B, 1, OUT_PAD), F32),
        compiler_params=pltpu.CompilerParams(
            dimension_semantics=("arbitrary",),
        ),
    )(exit_index, steps, data_t, tb, fb, embed_p, Wx, Wh, b2, bW, bb2,
      outW_p, outb_p)
    return out


def kernel(data, true_branch_nodes, false_branch_nodes, start_index,
           exit_index, steps, embed, Wx, Wh, b, branch_W, branch_b, out_W,
           out_b):
    del start_index
    data_t = jnp.transpose(data, (0, 2, 1))
    tb = true_branch_nodes.reshape(B, 1, N)
    fb = false_branch_nodes.reshape(B, 1, N)
    embed_p = jnp.pad(embed, ((0, VOCAB_PAD - VOCAB), (0, 0)))
    outW_p = jnp.pad(out_W, ((0, 0), (0, OUT_PAD - OUT_VOCAB)))
    outb_p = jnp.pad(out_b, (0, OUT_PAD - OUT_VOCAB)).reshape(1, OUT_PAD)
    b2 = b.reshape(1, 4 * H)
    bb2 = branch_b.reshape(1, 2)
    out = _forward_impl(data_t, tb, fb, exit_index, steps, embed_p, Wx, Wh,
                        b2, branch_W, bb2, outW_p, outb_p)
    return out[:, :, :OUT_VOCAB]


# speedup vs baseline: 21.5585x; 1.0010x over previous
import functools

import jax
import jax.numpy as jnp
from jax.experimental import pallas as pl
from jax.experimental.pallas import tpu as pltpu

B, N, L = 8, 128, 4
VOCAB, OUT_VOCAB, H = 1000, 1000, 128
VOCAB_PAD = 1024
OUT_PAD = 1024
MAX_STEPS = 6
F32 = jnp.float32


def _dot(a, b):
    return jax.lax.dot(a, b, preferred_element_type=F32)


def _fwd_kernel(exit_ref, steps_ref, data_ref, tidx_ref, fidx_ref, embed_ref,
                wx_ref, wh_ref, b_ref, bw_ref, bb_ref, outw_ref, outb_ref,
                out_ref):
    pid = pl.program_id(0)
    exit_i = exit_ref[pid]
    num_steps = steps_ref[pid]

    wh = wh_ref[...]
    bias = b_ref[...]

    col_iota = jax.lax.broadcasted_iota(jnp.int32, (N, VOCAB_PAD), 1)
    xw = []
    for l in range(L):
        toks = data_ref[0, l, :]
        oh = (toks[:, None] == col_iota).astype(F32)
        emb_l = _dot(oh, embed_ref[...])
        xw.append(_dot(emb_l, wx_ref[...]) + bias)

    row_iota = jax.lax.broadcasted_iota(jnp.int32, (N, N), 0)
    mt = (tidx_ref[0] == row_iota).astype(F32)
    mf = (fidx_ref[0] == row_iota).astype(F32)

    node_iota = jax.lax.broadcasted_iota(jnp.int32, (N, 1), 0)
    exit_mask = node_iota == exit_i
    ones = jnp.ones((N, N), F32)

    c = jnp.zeros((N, H), F32)
    h = jnp.zeros((N, H), F32)
    ip = (node_iota == 0).astype(F32)

    for s in range(MAX_STEPS):
        cc, hh = c, h
        for l in range(L):
            z = xw[l] + _dot(hh, wh)
            i_g = jax.nn.sigmoid(z[:, :H])
            f_g = jax.nn.sigmoid(z[:, H:2 * H])
            g_g = jnp.tanh(z[:, 2 * H:3 * H])
            o_g = jax.nn.sigmoid(z[:, 3 * H:])
            cc = f_g * cc + i_g * g_g
            hh = o_g * jnp.tanh(cc)
        ce = jnp.where(exit_mask, c, cc)
        he = jnp.where(exit_mask, h, hh)
        hcat = jnp.concatenate([ce, he], axis=1)
        bl = _dot(hcat, bw_ref[...]) + bb_ref[...]
        p_true = jax.nn.sigmoid(bl[:, 0:1] - bl[:, 1:2])
        wt = p_true * ip
        wf = (1.0 - p_true) * ip
        a = jnp.concatenate([hcat, ones], axis=1)
        r = _dot(mt, wt * a) + _dot(mf, wf * a)
        ip_new = r[:, 2 * H:2 * H + 1]
        denom = ip_new + 1e-7
        keep = jnp.int32(s) < num_steps
        c = jnp.where(keep, r[:, :H] / denom, c)
        h = jnp.where(keep, r[:, H:2 * H] / denom, h)
        ip = jnp.where(keep, ip_new, ip)

    e_row = (jax.lax.broadcasted_iota(jnp.int32, (1, N), 1) == exit_i
             ).astype(F32)
    fin = jnp.concatenate([_dot(e_row, c), _dot(e_row, h)], axis=1)
    out_ref[0] = _dot(fin, outw_ref[...]) + outb_ref[...]


@functools.partial(jax.jit, static_argnames=())
def _forward_impl(data_t, tb, fb, exit_index, steps, embed_p, Wx, Wh, b2,
                  bW, bb2, outW_p, outb_p):
    grid_spec = pltpu.PrefetchScalarGridSpec(
        num_scalar_prefetch=2,
        grid=(B,),
        in_specs=[
            pl.BlockSpec((1, L, N), lambda i, *_: (i, 0, 0)),
            pl.BlockSpec((1, 1, N), lambda i, *_: (i, 0, 0)),
            pl.BlockSpec((1, 1, N), lambda i, *_: (i, 0, 0)),
            pl.BlockSpec((VOCAB_PAD, H), lambda i, *_: (0, 0)),
            pl.BlockSpec((H, 4 * H), lambda i, *_: (0, 0)),
            pl.BlockSpec((H, 4 * H), lambda i, *_: (0, 0)),
            pl.BlockSpec((1, 4 * H), lambda i, *_: (0, 0)),
            pl.BlockSpec((2 * H, 2), lambda i, *_: (0, 0)),
            pl.BlockSpec((1, 2), lambda i, *_: (0, 0)),
            pl.BlockSpec((2 * H, OUT_PAD), lambda i, *_: (0, 0)),
            pl.BlockSpec((1, OUT_PAD), lambda i, *_: (0, 0)),
        ],
        out_specs=pl.BlockSpec((1, 1, OUT_PAD), lambda i, *_: (i, 0, 0)),
    )
    out = pl.pallas_call(
        _fwd_kernel,
        grid_spec=grid_spec,
        out_shape=jax.ShapeDtypeStruct((B, 1, OUT_PAD), F32),
        compiler_params=pltpu.CompilerParams(
            dimension_semantics=("parallel",),
        ),
    )(exit_index, steps, data_t, tb, fb, embed_p, Wx, Wh, b2, bW, bb2,
      outW_p, outb_p)
    return out


def kernel(data, true_branch_nodes, false_branch_nodes, start_index,
           exit_index, steps, embed, Wx, Wh, b, branch_W, branch_b, out_W,
           out_b):
    del start_index
    data_t = jnp.transpose(data, (0, 2, 1))
    tb = true_branch_nodes.reshape(B, 1, N)
    fb = false_branch_nodes.reshape(B, 1, N)
    embed_p = jnp.pad(embed, ((0, VOCAB_PAD - VOCAB), (0, 0)))
    outW_p = jnp.pad(out_W, ((0, 0), (0, OUT_PAD - OUT_VOCAB)))
    outb_p = jnp.pad(out_b, (0, OUT_PAD - OUT_VOCAB)).reshape(1, OUT_PAD)
    b2 = b.reshape(1, 4 * H)
    bb2 = branch_b.reshape(1, 2)
    out = _forward_impl(data_t, tb, fb, exit_index, steps, embed_p, Wx, Wh,
                        b2, branch_W, bb2, outW_p, outb_p)
    return out[:, :, :OUT_VOCAB]


# speedup vs baseline: 46.3579x; 2.1503x over previous
import functools

import jax
import jax.numpy as jnp
from jax.experimental import pallas as pl
from jax.experimental.pallas import tpu as pltpu

B, N, L = 8, 128, 4
VOCAB, OUT_VOCAB, H = 1000, 1000, 128
VOCAB_PAD = 1024
OUT_PAD = 1024
MAX_STEPS = 6
BN = B * N
F32 = jnp.float32


def _dot(a, b):
    return jax.lax.dot(a, b, preferred_element_type=F32)


def _sig(x):
    return 0.5 * jnp.tanh(0.5 * x) + 0.5


def _fwd_kernel(exit_ref, steps_ref, data_ref, tidx_ref, fidx_ref, embed_ref,
                wx_ref, wh_ref, b_ref, bw_ref, bb_ref, outw_ref, outb_ref,
                out_ref, xw_ref):
    wh = wh_ref[...]
    bias = b_ref[...]
    embed = embed_ref[...]
    wx = wx_ref[...]

    col_iota = jax.lax.broadcasted_iota(jnp.int32, (N, VOCAB_PAD), 1)
    for b in range(B):
        for l in range(L):
            toks = data_ref[b, l, :]
            oh = (toks[:, None] == col_iota).astype(F32)
            emb_l = _dot(oh, embed)
            xw_ref[l, b * N:(b + 1) * N, :] = _dot(emb_l, wx) + bias

    row_iota = jax.lax.broadcasted_iota(jnp.int32, (N, N), 0)
    mt = [(tidx_ref[b] == row_iota).astype(F32) for b in range(B)]
    mf = [(fidx_ref[b] == row_iota).astype(F32) for b in range(B)]

    node_iota = jax.lax.broadcasted_iota(jnp.int32, (N, 1), 0)
    exit_mask = jnp.concatenate(
        [node_iota == exit_ref[b] for b in range(B)], axis=0)
    steps_vec = jnp.concatenate(
        [jnp.full((N, 1), steps_ref[b], jnp.int32) for b in range(B)], axis=0)
    ip0 = (node_iota == 0).astype(F32)
    ones = jnp.ones((BN, N), F32)

    c = jnp.zeros((BN, H), F32)
    h = jnp.zeros((BN, H), F32)
    ip = jnp.concatenate([ip0] * B, axis=0)

    for s in range(MAX_STEPS):
        cc, hh = c, h
        for l in range(L):
            z = xw_ref[l] + _dot(hh, wh)
            i_g = _sig(z[:, :H])
            f_g = _sig(z[:, H:2 * H])
            g_g = jnp.tanh(z[:, 2 * H:3 * H])
            o_g = _sig(z[:, 3 * H:])
            cc = f_g * cc + i_g * g_g
            hh = o_g * jnp.tanh(cc)
        ce = jnp.where(exit_mask, c, cc)
        he = jnp.where(exit_mask, h, hh)
        hcat = jnp.concatenate([ce, he], axis=1)
        bl = _dot(hcat, bw_ref[...]) + bb_ref[...]
        p_true = _sig(bl[:, 0:1] - bl[:, 1:2])
        a = jnp.concatenate([hcat, ones], axis=1)
        at = (p_true * ip) * a
        af = ((1.0 - p_true) * ip) * a
        r = jnp.concatenate(
            [_dot(mt[b], at[b * N:(b + 1) * N]) +
             _dot(mf[b], af[b * N:(b + 1) * N]) for b in range(B)], axis=0)
        ip_new = r[:, 2 * H:2 * H + 1]
        inv = 1.0 / (ip_new + 1e-7)
        keep = jnp.int32(s) < steps_vec
        c = jnp.where(keep, r[:, :H] * inv, c)
        h = jnp.where(keep, r[:, H:2 * H] * inv, h)
        ip = jnp.where(keep, ip_new, ip)

    sel_iota = jax.lax.broadcasted_iota(jnp.int32, (B, BN), 1)
    targets = jnp.concatenate(
        [jnp.full((1, 1), N * b + exit_ref[b], jnp.int32) for b in range(B)],
        axis=0)
    e_mat = (sel_iota == targets).astype(F32)
    ch = jnp.concatenate([c, h], axis=1)
    fin = _dot(e_mat, ch)
    out_ref[...] = _dot(fin, outw_ref[...]) + outb_ref[...]


@jax.jit
def _forward_impl(data_t, tb, fb, exit_index, steps, embed_p, Wx, Wh, b2,
                  bW, bb2, outW_p, outb_p):
    grid_spec = pltpu.PrefetchScalarGridSpec(
        num_scalar_prefetch=2,
        grid=(1,),
        in_specs=[
            pl.BlockSpec((B, L, N), lambda i, *_: (0, 0, 0)),
            pl.BlockSpec((B, 1, N), lambda i, *_: (0, 0, 0)),
            pl.BlockSpec((B, 1, N), lambda i, *_: (0, 0, 0)),
            pl.BlockSpec((VOCAB_PAD, H), lambda i, *_: (0, 0)),
            pl.BlockSpec((H, 4 * H), lambda i, *_: (0, 0)),
            pl.BlockSpec((H, 4 * H), lambda i, *_: (0, 0)),
            pl.BlockSpec((1, 4 * H), lambda i, *_: (0, 0)),
            pl.BlockSpec((2 * H, 2), lambda i, *_: (0, 0)),
            pl.BlockSpec((1, 2), lambda i, *_: (0, 0)),
            pl.BlockSpec((2 * H, OUT_PAD), lambda i, *_: (0, 0)),
            pl.BlockSpec((1, OUT_PAD), lambda i, *_: (0, 0)),
        ],
        out_specs=pl.BlockSpec((B, OUT_PAD), lambda i, *_: (0, 0)),
        scratch_shapes=[pltpu.VMEM((L, BN, 4 * H), F32)],
    )
    out = pl.pallas_call(
        _fwd_kernel,
        grid_spec=grid_spec,
        out_shape=jax.ShapeDtypeStruct((B, OUT_PAD), F32),
        compiler_params=pltpu.CompilerParams(
            dimension_semantics=("arbitrary",),
        ),
    )(exit_index, steps, data_t, tb, fb, embed_p, Wx, Wh, b2, bW, bb2,
      outW_p, outb_p)
    return out


def kernel(data, true_branch_nodes, false_branch_nodes, start_index,
           exit_index, steps, embed, Wx, Wh, b, branch_W, branch_b, out_W,
           out_b):
    del start_index
    data_t = jnp.transpose(data, (0, 2, 1))
    tb = true_branch_nodes.reshape(B, 1, N)
    fb = false_branch_nodes.reshape(B, 1, N)
    embed_p = jnp.pad(embed, ((0, VOCAB_PAD - VOCAB), (0, 0)))
    outW_p = jnp.pad(out_W, ((0, 0), (0, OUT_PAD - OUT_VOCAB)))
    outb_p = jnp.pad(out_b, (0, OUT_PAD - OUT_VOCAB)).reshape(1, OUT_PAD)
    b2 = b.reshape(1, 4 * H)
    bb2 = branch_b.reshape(1, 2)
    out = _forward_impl(data_t, tb, fb, exit_index, steps, embed_p, Wx, Wh,
                        b2, branch_W, bb2, outW_p, outb_p)
    return out[:, None, :OUT_VOCAB]
